# baseline (speedup 1.0000x reference)
import functools

import jax
import jax.numpy as jnp
from jax.experimental import pallas as pl
from jax.experimental.pallas import tpu as pltpu


def _tinycnn_kernel(bt, x_ref, w1_ref, b1_ref, rs1_ref, p1_ref, k2c_ref,
                    b2_ref, rs2_ref, p2_ref, w3_ref, b3_ref, w4_ref, b4_ref,
                    out_ref, lhs1_ref, xp2_ref):
    f32, bf16 = jnp.float32, jnp.bfloat16
    B = bt
    G = B // 8

    xq = x_ref[...].reshape(B, 8, 128)
    z32 = jnp.zeros((B, 1, 32), bf16)
    lhs1_ref[:, :, 0:128] = xq
    lhs1_ref[:, 0:1, 128:160] = z32
    lhs1_ref[:, 1:8, 128:160] = xq[:, 0:7, 96:128]
    lhs1_ref[:, 7:8, 160:192] = z32
    lhs1_ref[:, 0:7, 160:192] = xq[:, 1:8, 0:32]

    lhs1 = lhs1_ref[...].reshape(B * 8, 192)
    acc1 = jnp.dot(lhs1, w1_ref[...], preferred_element_type=f32)
    a1 = jnp.maximum(acc1 + b1_ref[...], 0.0).astype(bf16)

    rs1 = rs1_ref[...]
    v1parts = []
    for q4 in range(G // 4):
        gparts = []
        for j in range(4):
            blk = a1[(4 * q4 + j) * 64:(4 * q4 + j + 1) * 64, :]
            gparts.append(jnp.concatenate(
                [blk[:, jj * 256:(jj + 1) * 256] for jj in range(4)], axis=0))
        a1q = jnp.concatenate(gparts, axis=1)
        r = jnp.dot(rs1, a1q, preferred_element_type=f32)
        vq = jnp.maximum(r[0:128, :], r[128:256, :]).astype(bf16)
        v1parts += [vq[:, j * 256:(j + 1) * 256] for j in range(4)]
    v1 = jnp.concatenate(v1parts, axis=0)

    q = jnp.dot(v1, p1_ref[...], preferred_element_type=f32)
    pooled1 = jnp.maximum(q[:, 0:128], q[:, 128:256]).astype(bf16)
    pr = pooled1.reshape(B, 16, 128)

    xp2_ref[:, 0:1, 0:128] = jnp.zeros((B, 1, 128), bf16)
    xp2_ref[:, 15:16, 256:384] = jnp.zeros((B, 1, 128), bf16)
    xp2_ref[:, 1:16, 0:128] = pr[:, 0:15, :]
    xp2_ref[:, :, 128:256] = pr
    xp2_ref[:, 0:15, 256:384] = pr[:, 1:16, :]

    lhs2 = xp2_ref[...].reshape(B * 16, 384)
    acc2 = jnp.dot(lhs2, k2c_ref[...], preferred_element_type=f32)
    a2 = jnp.maximum(acc2 + b2_ref[...], 0.0).astype(bf16)

    rs2 = rs2_ref[...]
    v2parts = []
    for q4 in range(G // 4):
        a2q = jnp.concatenate(
            [a2[(4 * q4 + j) * 128:(4 * q4 + j + 1) * 128, :]
             for j in range(4)], axis=1)
        r2 = jnp.dot(rs2, a2q, preferred_element_type=f32)
        vq = jnp.maximum(r2[0:64, :], r2[64:128, :]).astype(bf16)
        v2parts += [vq[:, j * 256:(j + 1) * 256] for j in range(4)]
    v2 = jnp.concatenate(v2parts, axis=0)

    q2 = jnp.dot(v2, p2_ref[...], preferred_element_type=f32)
    pooled2 = jnp.maximum(q2[:, 0:128], q2[:, 128:256]).astype(bf16)

    flat = pooled2.reshape(B, 1024)
    h = jnp.dot(flat, w3_ref[...], preferred_element_type=f32)
    h = jnp.maximum(h + b3_ref[...], 0.0)

    logits = jnp.dot(h.astype(bf16), w4_ref[...],
                     preferred_element_type=f32) + b4_ref[...]
    out_ref[...] = logits[:, 0:16]


def _const_spec(arr):
    if arr.ndim == 3:
        return pl.BlockSpec(arr.shape, lambda i: (0, 0, 0))
    return pl.BlockSpec(arr.shape, lambda i: (0, 0))


def _forward(x_packed, kparams, bt):
    n = x_packed.shape[0] // 8
    x = x_packed
    n_pad = ((n + bt - 1) // bt) * bt
    if n_pad != n:
        x = jnp.concatenate(
            [x, jnp.zeros(((n_pad - n) * 8, 128), x.dtype)], axis=0)

    in_specs = [pl.BlockSpec((bt * 8, 128), lambda i: (i, 0))]
    in_specs += [_const_spec(w) for w in kparams]

    out = pl.pallas_call(
        functools.partial(_tinycnn_kernel, bt),
        out_shape=jax.ShapeDtypeStruct((n_pad, 16), jnp.float32),
        grid=(n_pad // bt,),
        in_specs=in_specs,
        out_specs=pl.BlockSpec((bt, 16), lambda i: (i, 0)),
        scratch_shapes=[pltpu.VMEM((bt, 8, 192), jnp.bfloat16),
                        pltpu.VMEM((bt, 16, 384), jnp.bfloat16)],
        compiler_params=pltpu.CompilerParams(
            dimension_semantics=("parallel",)),
    )(x, *kparams)
    return out[:n, :10]


def kernel(x, K1, b1rep, RS1E, RS1O, P1A, P1B, K2, b2rep, RS2E, RS2O,
           P2A, P2B, W3, b3, W4, b4):
    n = x.shape[0]
    xp = x.reshape(n * 8, 128).astype(jnp.bfloat16)

    w1 = jnp.zeros((192, 1024), jnp.bfloat16)
    for j in range(4):
        for dy in range(3):
            ro = j + dy - 1
            if ro == -1:
                base = 128
            elif ro == 4:
                base = 160
            else:
                base = ro * 32
            w1 = w1.at[base:base + 32, j * 256:(j + 1) * 256].set(
                K1[dy * 32:(dy + 1) * 32, :])
    b1p = jnp.tile(b1rep, (1, 4))

    idx = [i * 32 + 4 * qq + j
           for j in range(4) for i in range(8) for qq in range(8)]
    rs1 = jnp.concatenate([RS1E, RS1O], axis=0)[:, jnp.array(idx)]

    p1 = jnp.concatenate([P1A, P1B], axis=1)
    rs2 = jnp.concatenate([RS2E, RS2O], axis=0)
    p2 = jnp.concatenate([P2A, P2B], axis=1)
    k2c = K2.reshape(384, 256)
    w3e = jnp.concatenate(
        [W3[:, ho * 128:ho * 128 + 64] for ho in range(8)], axis=0)
    kparams = (w1, b1p, rs1, p1, k2c, b2rep, rs2, p2, w3e, b3, W4, b4)
    return _forward(xp, kparams, 512)


# speedup vs baseline: 1.1093x; 1.1093x over previous
import functools

import jax
import jax.numpy as jnp
from jax.experimental import pallas as pl
from jax.experimental.pallas import tpu as pltpu


def _tinycnn_kernel(bt, x_ref, k1_ref, b1_ref, rs1_ref, p1_ref, k2c_ref,
                    b2_ref, rs2_ref, p2_ref, w3_ref, b3_ref, w4_ref, b4_ref,
                    out_ref, lhs1_ref, xp2_ref):
    f32, bf16 = jnp.float32, jnp.bfloat16
    B = bt
    G = B // 8

    xb = x_ref[...].reshape(B, 32, 32)
    zb = jnp.zeros((B, 1, 32), bf16)
    lhs1_ref[:, 0:1, 0:32] = zb
    lhs1_ref[:, 31:32, 64:96] = zb
    lhs1_ref[:, 1:32, 0:32] = xb[:, 0:31, :]
    lhs1_ref[:, :, 32:64] = xb
    lhs1_ref[:, 0:31, 64:96] = xb[:, 1:32, :]

    lhs1 = lhs1_ref[...].reshape(B * 32, 96)
    acc1 = jnp.dot(lhs1, k1_ref[...], preferred_element_type=f32)
    a1 = jnp.maximum(acc1 + b1_ref[...], 0.0).astype(bf16)

    rs1 = rs1_ref[...]
    v1parts = []
    for q4 in range(G // 4):
        a1q = jnp.concatenate(
            [a1[(4 * q4 + j) * 256:(4 * q4 + j + 1) * 256, :]
             for j in range(4)], axis=1)
        r = jnp.dot(rs1, a1q, preferred_element_type=f32)
        vq = jnp.maximum(r[0:128, :], r[128:256, :]).astype(bf16)
        v1parts += [vq[:, j * 256:(j + 1) * 256] for j in range(4)]
    v1 = jnp.concatenate(v1parts, axis=0)

    q = jnp.dot(v1, p1_ref[...], preferred_element_type=f32)
    pooled1 = jnp.maximum(q[:, 0:128], q[:, 128:256]).astype(bf16)
    pr = pooled1.reshape(B, 16, 128)

    xp2_ref[:, 0:1, 0:128] = jnp.zeros((B, 1, 128), bf16)
    xp2_ref[:, 15:16, 256:384] = jnp.zeros((B, 1, 128), bf16)
    xp2_ref[:, 1:16, 0:128] = pr[:, 0:15, :]
    xp2_ref[:, :, 128:256] = pr
    xp2_ref[:, 0:15, 256:384] = pr[:, 1:16, :]

    lhs2 = xp2_ref[...].reshape(B * 16, 384)
    acc2 = jnp.dot(lhs2, k2c_ref[...], preferred_element_type=f32)
    a2 = jnp.maximum(acc2 + b2_ref[...], 0.0).astype(bf16)

    rs2 = rs2_ref[...]
    v2parts = []
    for q4 in range(G // 4):
        a2q = jnp.concatenate(
            [a2[(4 * q4 + j) * 128:(4 * q4 + j + 1) * 128, :]
             for j in range(4)], axis=1)
        r2 = jnp.dot(rs2, a2q, preferred_element_type=f32)
        vq = jnp.maximum(r2[0:64, :], r2[64:128, :]).astype(bf16)
        v2parts += [vq[:, j * 256:(j + 1) * 256] for j in range(4)]
    v2 = jnp.concatenate(v2parts, axis=0)

    q2 = jnp.dot(v2, p2_ref[...], preferred_element_type=f32)
    pooled2 = jnp.maximum(q2[:, 0:128], q2[:, 128:256]).astype(bf16)

    flat = pooled2.reshape(B, 1024)
    h = jnp.dot(flat, w3_ref[...], preferred_element_type=f32)
    h = jnp.maximum(h + b3_ref[...], 0.0)

    logits = jnp.dot(h.astype(bf16), w4_ref[...],
                     preferred_element_type=f32) + b4_ref[...]
    out_ref[...] = logits[:, 0:16]


def _const_spec(arr):
    if arr.ndim == 3:
        return pl.BlockSpec(arr.shape, lambda i: (0, 0, 0))
    return pl.BlockSpec(arr.shape, lambda i: (0, 0))


def _forward(x_rows, kparams, bt):
    n = x_rows.shape[0] // 32
    x = x_rows
    n_pad = ((n + bt - 1) // bt) * bt
    if n_pad != n:
        x = jnp.concatenate(
            [x, jnp.zeros(((n_pad - n) * 32, 32), x.dtype)], axis=0)

    in_specs = [pl.BlockSpec((bt * 32, 32), lambda i: (i, 0))]
    in_specs += [_const_spec(w) for w in kparams]

    out = pl.pallas_call(
        functools.partial(_tinycnn_kernel, bt),
        out_shape=jax.ShapeDtypeStruct((n_pad, 16), jnp.float32),
        grid=(n_pad // bt,),
        in_specs=in_specs,
        out_specs=pl.BlockSpec((bt, 16), lambda i: (i, 0)),
        scratch_shapes=[pltpu.VMEM((bt, 32, 96), jnp.bfloat16),
                        pltpu.VMEM((bt, 16, 384), jnp.bfloat16)],
        compiler_params=pltpu.CompilerParams(
            dimension_semantics=("parallel",)),
    )(x, *kparams)
    return out[:n, :10]


def kernel(x, K1, b1rep, RS1E, RS1O, P1A, P1B, K2, b2rep, RS2E, RS2O,
           P2A, P2B, W3, b3, W4, b4):
    n = x.shape[0]
    xr = x.reshape(n * 32, 32).astype(jnp.bfloat16)
    rs1 = jnp.concatenate([RS1E, RS1O], axis=0)
    p1 = jnp.concatenate([P1A, P1B], axis=1)
    rs2 = jnp.concatenate([RS2E, RS2O], axis=0)
    p2 = jnp.concatenate([P2A, P2B], axis=1)
    k2c = K2.reshape(384, 256)
    w3e = jnp.concatenate(
        [W3[:, ho * 128:ho * 128 + 64] for ho in range(8)], axis=0)
    kparams = (K1, b1rep, rs1, p1, k2c, b2rep, rs2, p2, w3e, b3, W4, b4)
    return _forward(xr, kparams, 512)


# speedup vs baseline: 1.1152x; 1.0053x over previous
import functools

import jax
import jax.numpy as jnp
from jax.experimental import pallas as pl
from jax.experimental.pallas import tpu as pltpu


def _tinycnn_kernel(bt, x_ref, k1_ref, b1_ref, rs1_ref, p1_ref, k2c_ref,
                    b2_ref, rs2_ref, p2_ref, w3_ref, b3_ref, w4_ref, b4_ref,
                    out_ref, lhs1_ref, xp2_ref):
    f32, bf16 = jnp.float32, jnp.bfloat16
    B = bt
    G = B // 8

    xb = x_ref[...].reshape(B, 32, 32)
    zb = jnp.zeros((B, 1, 32), bf16)
    lhs1_ref[:, 0:1, 0:32] = zb
    lhs1_ref[:, 31:32, 64:96] = zb
    lhs1_ref[:, 1:32, 0:32] = xb[:, 0:31, :]
    lhs1_ref[:, :, 32:64] = xb
    lhs1_ref[:, 0:31, 64:96] = xb[:, 1:32, :]

    lhs1 = lhs1_ref[...].reshape(B * 32, 96)
    acc1 = jnp.dot(lhs1, k1_ref[...], preferred_element_type=f32)
    a1 = jnp.maximum(acc1 + b1_ref[...], 0.0).astype(bf16)

    rs1 = rs1_ref[...]
    v1parts = []
    for q4 in range(G // 4):
        a1q = jnp.concatenate(
            [a1[(4 * q4 + j) * 256:(4 * q4 + j + 1) * 256, :]
             for j in range(4)], axis=1)
        r = jnp.dot(rs1, a1q, preferred_element_type=f32)
        vq = jnp.maximum(r[0:128, :], r[128:256, :]).astype(bf16)
        v1parts += [vq[:, j * 256:(j + 1) * 256] for j in range(4)]
    v1 = jnp.concatenate(v1parts, axis=0)

    q = jnp.dot(v1, p1_ref[...], preferred_element_type=f32)
    pooled1 = jnp.maximum(q[:, 0:128], q[:, 128:256]).astype(bf16)
    pr = pooled1.reshape(B, 16, 128)

    xp2_ref[:, 0:1, 0:128] = jnp.zeros((B, 1, 128), bf16)
    xp2_ref[:, 15:16, 256:384] = jnp.zeros((B, 1, 128), bf16)
    xp2_ref[:, 1:16, 0:128] = pr[:, 0:15, :]
    xp2_ref[:, :, 128:256] = pr
    xp2_ref[:, 0:15, 256:384] = pr[:, 1:16, :]

    lhs2 = xp2_ref[...].reshape(B * 16, 384)
    acc2 = jnp.dot(lhs2, k2c_ref[...], preferred_element_type=f32)
    a2 = jnp.maximum(acc2 + b2_ref[...], 0.0).astype(bf16)

    rs2 = rs2_ref[...]
    v2parts = []
    for q4 in range(G // 4):
        a2q = jnp.concatenate(
            [a2[(4 * q4 + j) * 128:(4 * q4 + j + 1) * 128, :]
             for j in range(4)], axis=1)
        r2 = jnp.dot(rs2, a2q, preferred_element_type=f32)
        vq = jnp.maximum(r2[0:64, :], r2[64:128, :]).astype(bf16)
        v2parts += [vq[:, j * 256:(j + 1) * 256] for j in range(4)]
    v2 = jnp.concatenate(v2parts, axis=0)

    q2 = jnp.dot(v2, p2_ref[...], preferred_element_type=f32)
    pooled2 = jnp.maximum(q2[:, 0:128], q2[:, 128:256]).astype(bf16)

    flat = pooled2.reshape(B, 1024)
    h = jnp.dot(flat, w3_ref[...], preferred_element_type=f32)
    h = jnp.maximum(h + b3_ref[...], 0.0)

    logits = jnp.dot(h.astype(bf16), w4_ref[...],
                     preferred_element_type=f32) + b4_ref[...]
    out_ref[...] = logits[:, 0:10]


def _const_spec(arr):
    if arr.ndim == 3:
        return pl.BlockSpec(arr.shape, lambda i: (0, 0, 0))
    return pl.BlockSpec(arr.shape, lambda i: (0, 0))


def _forward(x_rows, kparams, bt):
    n = x_rows.shape[0] // 32
    x = x_rows
    n_pad = ((n + bt - 1) // bt) * bt
    if n_pad != n:
        x = jnp.concatenate(
            [x, jnp.zeros(((n_pad - n) * 32, 32), x.dtype)], axis=0)

    in_specs = [pl.BlockSpec((bt * 32, 32), lambda i: (i, 0))]
    in_specs += [_const_spec(w) for w in kparams]

    out = pl.pallas_call(
        functools.partial(_tinycnn_kernel, bt),
        out_shape=jax.ShapeDtypeStruct((n_pad, 10), jnp.float32),
        grid=(n_pad // bt,),
        in_specs=in_specs,
        out_specs=pl.BlockSpec((bt, 10), lambda i: (i, 0)),
        scratch_shapes=[pltpu.VMEM((bt, 32, 96), jnp.bfloat16),
                        pltpu.VMEM((bt, 16, 384), jnp.bfloat16)],
        compiler_params=pltpu.CompilerParams(
            dimension_semantics=("parallel",)),
    )(x, *kparams)
    return out[:n]


def kernel(x, K1, b1rep, RS1E, RS1O, P1A, P1B, K2, b2rep, RS2E, RS2O,
           P2A, P2B, W3, b3, W4, b4):
    n = x.shape[0]
    xr = x.reshape(n * 32, 32).astype(jnp.bfloat16)
    rs1 = jnp.concatenate([RS1E, RS1O], axis=0)
    p1 = jnp.concatenate([P1A, P1B], axis=1)
    rs2 = jnp.concatenate([RS2E, RS2O], axis=0)
    p2 = jnp.concatenate([P2A, P2B], axis=1)
    k2c = K2.reshape(384, 256)
    w3e = jnp.concatenate(
        [W3[:, ho * 128:ho * 128 + 64] for ho in range(8)], axis=0)
    kparams = (K1, b1rep, rs1, p1, k2c, b2rep, rs2, p2, w3e, b3, W4, b4)
    return _forward(xr, kparams, 512)
